# baseline (speedup 1.0000x reference)
import functools

import jax
import jax.numpy as jnp
from jax import lax
from jax.experimental import pallas as pl
from jax.experimental.pallas import tpu as pltpu
from jax.experimental.pallas import tpu_sc as plsc

BN = 2
NP = 2048
DM = 256
KN = 17
RB = 128
XP = 128
TW = 2 * DM + XP
NS = 4
BPB = NP // RB
NBLK = BN * NP // RB
TOT = BN * NP
GROWS = KN * TOT


def _mm(a, w):
    return lax.dot_general(a, w, dimension_numbers=(((1,), (1,)), ((), ())),
                           preferred_element_type=jnp.float32)


def _projknn_body(ssr, ssn, xaaug, feat, xyzp, W1, b1, Wq, Wk, Wv,
                  q_o, tab_o, idx_o):
    g = pl.program_id(0)
    b = g // BPB
    x = _mm(feat[...], W1[...]) + b1[...]
    q_o[...] = _mm(x, Wq[...])
    kp = _mm(x, Wk[...])
    vp = _mm(x, Wv[...])
    tab_o[...] = jnp.concatenate([kp, vp], axis=1).astype(jnp.bfloat16)
    dt = _mm(xyzp[...], xaaug[0])
    d = (ssr[...] - 2.0 * dt) + ssn[0]
    sr = RB // NS
    lanes = lax.broadcasted_iota(jnp.int32, (sr, NP), 1).astype(jnp.float32)
    ds = [d[s * sr:(s + 1) * sr] for s in range(NS)]
    colss = [[] for _ in range(NS)]
    for _ in range(KN):
        for s in range(NS):
            m = jnp.min(ds[s], axis=1, keepdims=True)
            hit = ds[s] <= m
            idxj = jnp.min(jnp.where(hit, lanes, 1.0 * NP), axis=1,
                           keepdims=True)
            ds[s] = jnp.where(hit, 1e30, ds[s])
            colss[s].append(idxj)
    for s in range(NS):
        idx = jnp.concatenate(colss[s], axis=1).astype(jnp.int32)
        idx_o[s * sr:(s + 1) * sr, :] = idx + b * NP


def _attn_body(q, kvg, xyg, xyzp, feat, Wd1p, bd1, Wd2, bd2,
               Wg1, bg1, Wg2, bg2, W2, b2, out_o):
    bf16 = jnp.bfloat16
    f32 = jnp.float32
    kv = kvg[...].reshape(KN * RB, 2 * DM)
    kk = kv[:, :DM].astype(f32)
    vvpos_src = kv[:, DM:].astype(f32)
    nx = xyg[...].reshape(KN * RB, XP)
    xt = jnp.concatenate([xyzp[...]] * KN, axis=0)
    qt = jnp.concatenate([q[...]] * KN, axis=0)
    delta = xt - nx
    pe1 = jax.nn.relu(_mm(delta, Wd1p[...]) + bd1[...])
    pos = _mm(pe1.astype(bf16), Wd2[...].astype(bf16)) + bd2[...]
    h = qt - kk + pos
    a1 = jax.nn.relu(_mm(h.astype(bf16), Wg1[...].astype(bf16)) + bg1[...])
    att = (_mm(a1.astype(bf16), Wg2[...].astype(bf16)) + bg2[...]) * (1.0 / 16.0)
    vp = vvpos_src + pos
    m = att[0:RB]
    for j in range(1, KN):
        m = jnp.maximum(m, att[j * RB:(j + 1) * RB])
    s = jnp.zeros((RB, DM), jnp.float32)
    num = jnp.zeros((RB, DM), jnp.float32)
    for j in range(KN):
        e = jnp.exp(att[j * RB:(j + 1) * RB] - m)
        s = s + e
        num = num + e * vp[j * RB:(j + 1) * RB]
    res = num / s
    out_o[...] = _mm(res, W2[...]) + b2[...] + feat[...]


def _gather(kvtab, xyztab, idxg):
    info = plsc.get_sparse_core_info()
    nw = info.num_cores * info.num_subcores
    grows = idxg.size
    per_w = grows // nw
    ch = 64
    nch = per_w // ch
    mesh = plsc.VectorSubcoreMesh(core_axis_name="c", subcore_axis_name="s")

    @functools.partial(
        pl.kernel, mesh=mesh,
        out_type=[
            jax.ShapeDtypeStruct((grows, 2 * DM // 2), jnp.int32),
            jax.ShapeDtypeStruct((grows, XP), jnp.float32),
        ],
        scratch_types=[
            pltpu.VMEM((nch, ch), jnp.int32),
            pltpu.VMEM((ch, 2 * DM // 2), jnp.int32),
            pltpu.VMEM((ch, 2 * DM // 2), jnp.int32),
            pltpu.VMEM((ch, XP), jnp.float32),
            pltpu.VMEM((ch, XP), jnp.float32),
            pltpu.SemaphoreType.DMA,
            pltpu.SemaphoreType.DMA,
            pltpu.SemaphoreType.DMA,
            pltpu.SemaphoreType.DMA,
        ],
    )
    def gk(kv_h, xyz_h, idx_h, okv_h, oxy_h, idx_v,
           kv0, kv1, xy0, xy1, skv0, skv1, sxy0, sxy1):
        wid = lax.axis_index("s") * info.num_cores + lax.axis_index("c")
        base = wid * per_w
        pltpu.sync_copy(idx_h.at[wid], idx_v)
        kvb = (kv0, kv1)
        xyb = (xy0, xy1)
        skv = (skv0, skv1)
        sxy = (sxy0, sxy1)

        def issue(c, slot):
            pltpu.async_copy(kv_h.at[idx_v.at[c]], kvb[slot], skv[slot])
            pltpu.async_copy(xyz_h.at[idx_v.at[c]], xyb[slot], sxy[slot])

        def drain(c, slot):
            pltpu.make_async_copy(kv_h.at[idx_v.at[c]], kvb[slot],
                                  skv[slot]).wait()
            pltpu.make_async_copy(xyz_h.at[idx_v.at[c]], xyb[slot],
                                  sxy[slot]).wait()
            pltpu.sync_copy(kvb[slot], okv_h.at[pl.ds(base + c * ch, ch)])
            pltpu.sync_copy(xyb[slot], oxy_h.at[pl.ds(base + c * ch, ch)])

        issue(0, 0)
        issue(1, 1)

        def body(g, carry):
            for bslot in range(2):
                c = 2 * g + bslot
                drain(c, bslot)

                @pl.when(c + 2 < nch)
                def _():
                    issue(c + 2, bslot)
            return carry

        lax.fori_loop(0, nch // 2, body, 0)
        if nch % 2:
            drain(nch - 1, (nch - 1) % 2)

    npts = kvtab.shape[0]
    kv32 = lax.bitcast_convert_type(
        kvtab.reshape(npts, DM, 2), jnp.int32)
    okv, oxy = gk(kv32, xyztab, idxg.reshape(nw, nch, ch))
    okv = lax.bitcast_convert_type(okv, jnp.bfloat16).reshape(grows, 2 * DM)
    return okv, oxy


def _prep(xyzf):
    f32 = jnp.float32
    ss = jnp.sum(xyzf * xyzf, axis=1, keepdims=True)
    xyzp = jnp.concatenate([xyzf, jnp.zeros((NP, XP - 3), f32)], axis=1)
    xaaug = xyzp.reshape(1, NP, XP)
    ssn = ss.reshape(1, 1, NP)
    return ss, ssn, xaaug, xyzp


_wspec = pl.BlockSpec((DM, DM), lambda g: (0, 0))
_bspec = pl.BlockSpec((1, DM), lambda g: (0, 0))
_rspec = pl.BlockSpec((RB, DM), lambda g: (g, 0))
_xspec = pl.BlockSpec((RB, XP), lambda g: (g, 0))


def _stage1(ssr, ssn, xaaug, featf, xyzp, W1, b1r, Wq, Wk, Wv):
    f32 = jnp.float32
    wspec, bspec, rspec, xspec = _wspec, _bspec, _rspec, _xspec
    return pl.pallas_call(
        _projknn_body,
        grid=(BPB,),
        in_specs=[
            pl.BlockSpec((RB, 1), lambda g: (g, 0)),
            pl.BlockSpec((1, 1, NP), lambda g: (g // BPB, 0, 0)),
            pl.BlockSpec((1, NP, XP), lambda g: (g // BPB, 0, 0)),
            rspec,
            xspec,
            wspec, bspec, wspec, wspec, wspec,
        ],
        out_specs=[
            rspec,
            pl.BlockSpec((RB, 2 * DM), lambda g: (g, 0)),
            pl.BlockSpec((RB, KN), lambda g: (g, 0)),
        ],
        out_shape=[
            jax.ShapeDtypeStruct((NP, DM), f32),
            jax.ShapeDtypeStruct((NP, 2 * DM), jnp.bfloat16),
            jax.ShapeDtypeStruct((NP, KN), jnp.int32),
        ],
    )(ssr, ssn, xaaug, featf, xyzp, W1, b1r, Wq, Wk, Wv)


def _stage2(q, kvg, xyg, xyzp, featf, Wd1p, bd1r, Wd2, bd2r,
            Wg1, bg1r, Wg2, bg2r, W2, b2r):
    f32 = jnp.float32
    wspec, bspec, rspec, xspec = _wspec, _bspec, _rspec, _xspec
    return pl.pallas_call(
        _attn_body,
        grid=(BPB,),
        in_specs=[
            rspec,
            pl.BlockSpec((KN, RB, 2 * DM), lambda g: (0, g, 0)),
            pl.BlockSpec((KN, RB, XP), lambda g: (0, g, 0)),
            xspec,
            rspec,
            pl.BlockSpec((DM, XP), lambda g: (0, 0)),
            bspec, wspec, bspec, wspec, bspec, wspec, bspec,
            wspec, bspec,
        ],
        out_specs=rspec,
        out_shape=jax.ShapeDtypeStruct((NP, DM), f32),
    )(q, kvg, xyg, xyzp, featf, Wd1p, bd1r, Wd2, bd2r,
      Wg1, bg1r, Wg2, bg2r, W2, b2r)


def kernel(xyz, features, W1, b1, W2, b2, Wq, Wk, Wv,
           Wd1, bd1, Wd2, bd2, Wg1, bg1, Wg2, bg2):
    f32 = jnp.float32
    Wd1p = jnp.concatenate([Wd1, jnp.zeros((DM, XP - 3), f32)], axis=1)
    b1r, b2r, bd1r, bd2r, bg1r, bg2r = (
        v.reshape(1, DM) for v in (b1, b2, bd1, bd2, bg1, bg2))

    outs = []
    for b in range(BN):
        featf = features[b]
        ssr, ssn, xaaug, xyzp = _prep(xyz[b].astype(f32))
        q, table, idxpm = _stage1(ssr, ssn, xaaug, featf, xyzp,
                                  W1, b1r, Wq, Wk, Wv)
        idxg = idxpm.T.reshape(KN * NP)
        kvg, xyg = _gather(table, xyzp, idxg)
        kvg = kvg.reshape(KN, NP, 2 * DM)
        xyg = xyg.reshape(KN, NP, XP)
        outs.append(_stage2(q, kvg, xyg, xyzp, featf, Wd1p, bd1r, Wd2, bd2r,
                            Wg1, bg1r, Wg2, bg2r, W2, b2r))
    return jnp.stack(outs)


# speedup vs baseline: 3.1847x; 3.1847x over previous
import functools

import jax
import jax.numpy as jnp
from jax import lax
from jax.experimental import pallas as pl
from jax.experimental.pallas import tpu as pltpu
from jax.experimental.pallas import tpu_sc as plsc

BN = 2
NP = 2048
DM = 256
KN = 17
RB = 128
XP = 128
TW = 2 * DM + XP
NS = 4
BPB = NP // RB
NBLK = BN * NP // RB
TOT = BN * NP
GROWS = KN * TOT


def _mm(a, w):
    return lax.dot_general(a, w, dimension_numbers=(((1,), (1,)), ((), ())),
                           preferred_element_type=jnp.float32)


def _projknn_body(ssr, ssn, xaaug, feat, xyzp, W1, b1, Wq, Wk, Wv,
                  q_o, tab_o, idx_o):
    g = pl.program_id(0)
    b = g // BPB
    x = _mm(feat[...], W1[...]) + b1[...]
    q_o[...] = _mm(x, Wq[...])
    kp = _mm(x, Wk[...])
    vp = _mm(x, Wv[...])
    def rbf(a):
        u = lax.bitcast_convert_type(a, jnp.int32)
        return (u + 0x7FFF + ((u >> 16) & 1)) >> 16
    tab_o[...] = (rbf(kp) << 16) | (rbf(vp) & 0xFFFF)
    dt = _mm(xyzp[...], xaaug[0])
    d = (ssr[...] - 2.0 * dt) + ssn[0]
    sr = RB // NS
    lanes = lax.broadcasted_iota(jnp.int32, (sr, NP), 1).astype(jnp.float32)
    ds = [d[s * sr:(s + 1) * sr] for s in range(NS)]
    colss = [[] for _ in range(NS)]
    for _ in range(KN):
        for s in range(NS):
            m = jnp.min(ds[s], axis=1, keepdims=True)
            hit = ds[s] <= m
            idxj = jnp.min(jnp.where(hit, lanes, 1.0 * NP), axis=1,
                           keepdims=True)
            ds[s] = jnp.where(hit, 1e30, ds[s])
            colss[s].append(idxj)
    for s in range(NS):
        idx = jnp.concatenate(colss[s], axis=1).astype(jnp.int32)
        idx_o[s * sr:(s + 1) * sr, :] = idx + b * NP


def _attn_body(q, kvg, xyg, xyzp, feat, Wd1p, bd1, Wd2, bd2,
               Wg1, bg1, Wg2, bg2, W2, b2, out_o):
    bf16 = jnp.bfloat16
    f32 = jnp.float32
    kv = kvg[...].reshape(KN * RB, DM)
    kk = lax.bitcast_convert_type(kv & jnp.int32(-65536), f32)
    vvpos_src = lax.bitcast_convert_type(kv << 16, f32)
    nx = xyg[...].reshape(KN * RB, XP)
    xt = jnp.concatenate([xyzp[...]] * KN, axis=0)
    qt = jnp.concatenate([q[...]] * KN, axis=0)
    delta = xt - nx
    pe1 = jax.nn.relu(_mm(delta, Wd1p[...]) + bd1[...])
    pos = _mm(pe1.astype(bf16), Wd2[...].astype(bf16)) + bd2[...]
    h = qt - kk + pos
    a1 = jax.nn.relu(_mm(h.astype(bf16), Wg1[...].astype(bf16)) + bg1[...])
    att = (_mm(a1.astype(bf16), Wg2[...].astype(bf16)) + bg2[...]) * (1.0 / 16.0)
    vp = vvpos_src + pos
    m = att[0:RB]
    for j in range(1, KN):
        m = jnp.maximum(m, att[j * RB:(j + 1) * RB])
    s = jnp.zeros((RB, DM), jnp.float32)
    num = jnp.zeros((RB, DM), jnp.float32)
    for j in range(KN):
        e = jnp.exp(att[j * RB:(j + 1) * RB] - m)
        s = s + e
        num = num + e * vp[j * RB:(j + 1) * RB]
    res = num / s
    out_o[...] = _mm(res, W2[...]) + b2[...] + feat[...]


def _gather(kvtab, xyztab, idxg):
    info = plsc.get_sparse_core_info()
    nw = info.num_cores * info.num_subcores
    grows = idxg.size
    per_w = grows // nw
    ch = 64
    nch = per_w // ch
    mesh = plsc.VectorSubcoreMesh(core_axis_name="c", subcore_axis_name="s")

    @functools.partial(
        pl.kernel, mesh=mesh,
        out_type=[
            jax.ShapeDtypeStruct((grows, DM), jnp.int32),
            jax.ShapeDtypeStruct((grows, XP), jnp.float32),
        ],
        scratch_types=[
            pltpu.VMEM((nch, ch), jnp.int32),
            pltpu.VMEM((ch, DM), jnp.int32),
            pltpu.VMEM((ch, DM), jnp.int32),
            pltpu.VMEM((ch, XP), jnp.float32),
            pltpu.VMEM((ch, XP), jnp.float32),
            pltpu.SemaphoreType.DMA,
            pltpu.SemaphoreType.DMA,
            pltpu.SemaphoreType.DMA,
            pltpu.SemaphoreType.DMA,
        ],
    )
    def gk(kv_h, xyz_h, idx_h, okv_h, oxy_h, idx_v,
           kv0, kv1, xy0, xy1, skv0, skv1, sxy0, sxy1):
        wid = lax.axis_index("s") * info.num_cores + lax.axis_index("c")
        base = wid * per_w
        pltpu.sync_copy(idx_h.at[wid], idx_v)
        kvb = (kv0, kv1)
        xyb = (xy0, xy1)
        skv = (skv0, skv1)
        sxy = (sxy0, sxy1)

        def issue(c, slot):
            pltpu.async_copy(kv_h.at[idx_v.at[c]], kvb[slot], skv[slot])
            pltpu.async_copy(xyz_h.at[idx_v.at[c]], xyb[slot], sxy[slot])

        def drain(c, slot):
            pltpu.make_async_copy(kv_h.at[idx_v.at[c]], kvb[slot],
                                  skv[slot]).wait()
            pltpu.make_async_copy(xyz_h.at[idx_v.at[c]], xyb[slot],
                                  sxy[slot]).wait()
            pltpu.sync_copy(kvb[slot], okv_h.at[pl.ds(base + c * ch, ch)])
            pltpu.sync_copy(xyb[slot], oxy_h.at[pl.ds(base + c * ch, ch)])

        issue(0, 0)
        issue(1, 1)

        def body(g, carry):
            for bslot in range(2):
                c = 2 * g + bslot
                drain(c, bslot)

                @pl.when(c + 2 < nch)
                def _():
                    issue(c + 2, bslot)
            return carry

        lax.fori_loop(0, nch // 2, body, 0)
        if nch % 2:
            drain(nch - 1, (nch - 1) % 2)

    return gk(kvtab, xyztab, idxg.reshape(nw, nch, ch))


def _prep(xyzf):
    f32 = jnp.float32
    ss = jnp.sum(xyzf * xyzf, axis=1, keepdims=True)
    xyzp = jnp.concatenate([xyzf, jnp.zeros((NP, XP - 3), f32)], axis=1)
    xaaug = xyzp.reshape(1, NP, XP)
    ssn = ss.reshape(1, 1, NP)
    return ss, ssn, xaaug, xyzp


_wspec = pl.BlockSpec((DM, DM), lambda g: (0, 0))
_bspec = pl.BlockSpec((1, DM), lambda g: (0, 0))
_rspec = pl.BlockSpec((RB, DM), lambda g: (g, 0))
_xspec = pl.BlockSpec((RB, XP), lambda g: (g, 0))


def _stage1(ssr, ssn, xaaug, featf, xyzp, W1, b1r, Wq, Wk, Wv):
    f32 = jnp.float32
    wspec, bspec, rspec, xspec = _wspec, _bspec, _rspec, _xspec
    return pl.pallas_call(
        _projknn_body,
        grid=(BPB,),
        in_specs=[
            pl.BlockSpec((RB, 1), lambda g: (g, 0)),
            pl.BlockSpec((1, 1, NP), lambda g: (g // BPB, 0, 0)),
            pl.BlockSpec((1, NP, XP), lambda g: (g // BPB, 0, 0)),
            rspec,
            xspec,
            wspec, bspec, wspec, wspec, wspec,
        ],
        out_specs=[
            rspec,
            pl.BlockSpec((RB, DM), lambda g: (g, 0)),
            pl.BlockSpec((RB, KN), lambda g: (g, 0)),
        ],
        out_shape=[
            jax.ShapeDtypeStruct((NP, DM), f32),
            jax.ShapeDtypeStruct((NP, DM), jnp.int32),
            jax.ShapeDtypeStruct((NP, KN), jnp.int32),
        ],
    )(ssr, ssn, xaaug, featf, xyzp, W1, b1r, Wq, Wk, Wv)


def _stage2(q, kvg, xyg, xyzp, featf, Wd1p, bd1r, Wd2, bd2r,
            Wg1, bg1r, Wg2, bg2r, W2, b2r):
    f32 = jnp.float32
    wspec, bspec, rspec, xspec = _wspec, _bspec, _rspec, _xspec
    return pl.pallas_call(
        _attn_body,
        grid=(BPB,),
        in_specs=[
            rspec,
            pl.BlockSpec((KN, RB, DM), lambda g: (0, g, 0)),
            pl.BlockSpec((KN, RB, XP), lambda g: (0, g, 0)),
            xspec,
            rspec,
            pl.BlockSpec((DM, XP), lambda g: (0, 0)),
            bspec, wspec, bspec, wspec, bspec, wspec, bspec,
            wspec, bspec,
        ],
        out_specs=rspec,
        out_shape=jax.ShapeDtypeStruct((NP, DM), f32),
    )(q, kvg, xyg, xyzp, featf, Wd1p, bd1r, Wd2, bd2r,
      Wg1, bg1r, Wg2, bg2r, W2, b2r)


def kernel(xyz, features, W1, b1, W2, b2, Wq, Wk, Wv,
           Wd1, bd1, Wd2, bd2, Wg1, bg1, Wg2, bg2):
    f32 = jnp.float32
    Wd1p = jnp.concatenate([Wd1, jnp.zeros((DM, XP - 3), f32)], axis=1)
    b1r, b2r, bd1r, bd2r, bg1r, bg2r = (
        v.reshape(1, DM) for v in (b1, b2, bd1, bd2, bg1, bg2))

    outs = []
    for b in range(BN):
        featf = features[b]
        ssr, ssn, xaaug, xyzp = _prep(xyz[b].astype(f32))
        q, table, idxpm = _stage1(ssr, ssn, xaaug, featf, xyzp,
                                  W1, b1r, Wq, Wk, Wv)
        idxg = idxpm.T.reshape(KN * NP)
        kvg, xyg = _gather(table, xyzp, idxg)
        kvg = kvg.reshape(KN, NP, DM)
        xyg = xyg.reshape(KN, NP, XP)
        outs.append(_stage2(q, kvg, xyg, xyzp, featf, Wd1p, bd1r, Wd2, bd2r,
                            Wg1, bg1r, Wg2, bg2r, W2, b2r))
    return jnp.stack(outs)
